# baseline (speedup 1.0000x reference)
import jax
import jax.numpy as jnp
from jax import lax
from jax.experimental import pallas as pl
from jax.experimental.pallas import tpu as pltpu

NUM_EVENTS = 5
NUM_INTERVALS = 20
BATCH = 16384
NUM_TARGETS = 128
STATE_WEIGHT = 1.0
SURVIVAL_WEIGHT = 1.0

NB = 16
ROWS_BLK = BATCH // NB


def _tc_body(sp_ref, st_ref, sm_ref, hz_ref, t_ref, ind_ref, out_ref,
             acc_ref, tT_ref, indT_ref):
    b = pl.program_id(0)
    e = pl.program_id(1)

    @pl.when(jnp.logical_and(b == 0, e == 0))
    def _init():
        acc_ref[0] = 0.0
        acc_ref[1] = 0.0
        acc_ref[2] = 0.0

    @pl.when(e == 0)
    def _mse():
        d = sp_ref[...] - st_ref[...]
        sm = sm_ref[...]
        acc_ref[0] = acc_ref[0] + jnp.sum(d * d * sm)
        acc_ref[1] = acc_ref[1] + jnp.sum(sm)
        tT_ref[...] = lax.transpose(t_ref[...], (1, 0))
        indT_ref[...] = lax.transpose(ind_ref[...], (1, 0))

    x = lax.transpose(hz_ref[...].reshape(ROWS_BLK, NUM_INTERVALS), (1, 0))
    tt = tT_ref[pl.ds(e, 1), :]
    ind = indT_ref[pl.ds(e, 1), :]

    jj = lax.broadcasted_iota(jnp.int32, (NUM_INTERVALS, ROWS_BLK), 0)
    bj = (jj.astype(jnp.float32) + 1.0) * 0.5
    cmp = tt > bj

    ex = jnp.exp(-x)
    p = 1.0 / (1.0 + ex)
    l1mp = jnp.log((1.0 - p) + 1e-8)
    mask_a = jnp.logical_and(cmp, jj < NUM_INTERVALS - 1)
    s_surv = jnp.sum(jnp.where(mask_a, l1mp, 0.0))

    idx = jnp.sum(cmp.astype(jnp.int32), axis=0, keepdims=True)
    idxc = jnp.minimum(idx, NUM_INTERVALS - 1)
    sel_b = jj == idxc
    xg = jnp.sum(jnp.where(sel_b, x, 0.0), axis=0, keepdims=True)
    pg = 1.0 / (1.0 + jnp.exp(-xg))
    lp = jnp.log(pg + 1e-8)
    s_haz = jnp.sum(jnp.where(ind > 0.5, lp, 0.0))

    acc_ref[2] = acc_ref[2] + (s_surv + s_haz)

    @pl.when(jnp.logical_and(b == NB - 1, e == NUM_EVENTS - 1))
    def _fin():
        state_loss = acc_ref[0] / (acc_ref[1] + 1e-8)
        surv_loss = -acc_ref[2] / jnp.float32(NUM_EVENTS * BATCH)
        out_ref[0, 0] = STATE_WEIGHT * state_loss + SURVIVAL_WEIGHT * surv_loss


def kernel(state_pred, hazard_logits, state_target, state_mask,
           event_times, event_indicators):
    out = pl.pallas_call(
        _tc_body,
        grid=(NB, NUM_EVENTS),
        in_specs=[
            pl.BlockSpec((ROWS_BLK, NUM_TARGETS), lambda b, e: (b, 0)),
            pl.BlockSpec((ROWS_BLK, NUM_TARGETS), lambda b, e: (b, 0)),
            pl.BlockSpec((ROWS_BLK, NUM_TARGETS), lambda b, e: (b, 0)),
            pl.BlockSpec((1, ROWS_BLK, NUM_INTERVALS), lambda b, e: (e, b, 0)),
            pl.BlockSpec((ROWS_BLK, NUM_EVENTS), lambda b, e: (b, 0)),
            pl.BlockSpec((ROWS_BLK, NUM_EVENTS), lambda b, e: (b, 0)),
        ],
        out_specs=pl.BlockSpec(memory_space=pltpu.SMEM),
        out_shape=jax.ShapeDtypeStruct((1, 1), jnp.float32),
        scratch_shapes=[
            pltpu.SMEM((4,), jnp.float32),
            pltpu.VMEM((NUM_EVENTS, ROWS_BLK), jnp.float32),
            pltpu.VMEM((NUM_EVENTS, ROWS_BLK), jnp.float32),
        ],
    )(state_pred, state_target, state_mask, hazard_logits,
      event_times, event_indicators)
    return out[0, 0]


# speedup vs baseline: 1.4159x; 1.4159x over previous
import jax
import jax.numpy as jnp
from jax import lax
from jax.experimental import pallas as pl
from jax.experimental.pallas import tpu as pltpu

NUM_EVENTS = 5
NUM_INTERVALS = 20
BATCH = 16384
NUM_TARGETS = 128
STATE_WEIGHT = 1.0
SURVIVAL_WEIGHT = 1.0

NB = 4
ROWS_BLK = BATCH // NB


def _tc_body(sp_ref, st_ref, sm_ref, hz_ref, t_ref, ind_ref, out_ref,
             acc_ref, tT_ref, indT_ref):
    b = pl.program_id(0)
    e = pl.program_id(1)

    @pl.when(jnp.logical_and(b == 0, e == 0))
    def _init():
        acc_ref[0] = 0.0
        acc_ref[1] = 0.0
        acc_ref[2] = 0.0

    @pl.when(e == 0)
    def _mse():
        d = sp_ref[...] - st_ref[...]
        sm = sm_ref[...]
        acc_ref[0] = acc_ref[0] + jnp.sum(d * d * sm)
        acc_ref[1] = acc_ref[1] + jnp.sum(sm)
        tT_ref[...] = lax.transpose(t_ref[...], (1, 0))
        indT_ref[...] = lax.transpose(ind_ref[...], (1, 0))

    x = lax.transpose(hz_ref[...].reshape(ROWS_BLK, NUM_INTERVALS), (1, 0))
    tt = tT_ref[pl.ds(e, 1), :]
    ind = indT_ref[pl.ds(e, 1), :]

    jj = lax.broadcasted_iota(jnp.int32, (NUM_INTERVALS, ROWS_BLK), 0)
    bj = (jj.astype(jnp.float32) + 1.0) * 0.5
    cmp = tt > bj

    ex = jnp.exp(-x)
    p = 1.0 / (1.0 + ex)
    l1mp = jnp.log((1.0 - p) + 1e-8)
    mask_a = jnp.logical_and(cmp, jj < NUM_INTERVALS - 1)
    s_surv = jnp.sum(jnp.where(mask_a, l1mp, 0.0))

    idx = jnp.sum(cmp.astype(jnp.int32), axis=0, keepdims=True)
    idxc = jnp.minimum(idx, NUM_INTERVALS - 1)
    sel_b = jj == idxc
    xg = jnp.sum(jnp.where(sel_b, x, 0.0), axis=0, keepdims=True)
    pg = 1.0 / (1.0 + jnp.exp(-xg))
    lp = jnp.log(pg + 1e-8)
    s_haz = jnp.sum(jnp.where(ind > 0.5, lp, 0.0))

    acc_ref[2] = acc_ref[2] + (s_surv + s_haz)

    @pl.when(jnp.logical_and(b == NB - 1, e == NUM_EVENTS - 1))
    def _fin():
        state_loss = acc_ref[0] / (acc_ref[1] + 1e-8)
        surv_loss = -acc_ref[2] / jnp.float32(NUM_EVENTS * BATCH)
        out_ref[0, 0] = STATE_WEIGHT * state_loss + SURVIVAL_WEIGHT * surv_loss


def kernel(state_pred, hazard_logits, state_target, state_mask,
           event_times, event_indicators):
    out = pl.pallas_call(
        _tc_body,
        grid=(NB, NUM_EVENTS),
        in_specs=[
            pl.BlockSpec((ROWS_BLK, NUM_TARGETS), lambda b, e: (b, 0)),
            pl.BlockSpec((ROWS_BLK, NUM_TARGETS), lambda b, e: (b, 0)),
            pl.BlockSpec((ROWS_BLK, NUM_TARGETS), lambda b, e: (b, 0)),
            pl.BlockSpec((1, ROWS_BLK, NUM_INTERVALS), lambda b, e: (e, b, 0)),
            pl.BlockSpec((ROWS_BLK, NUM_EVENTS), lambda b, e: (b, 0)),
            pl.BlockSpec((ROWS_BLK, NUM_EVENTS), lambda b, e: (b, 0)),
        ],
        out_specs=pl.BlockSpec(memory_space=pltpu.SMEM),
        out_shape=jax.ShapeDtypeStruct((1, 1), jnp.float32),
        scratch_shapes=[
            pltpu.SMEM((4,), jnp.float32),
            pltpu.VMEM((NUM_EVENTS, ROWS_BLK), jnp.float32),
            pltpu.VMEM((NUM_EVENTS, ROWS_BLK), jnp.float32),
        ],
    )(state_pred, state_target, state_mask, hazard_logits,
      event_times, event_indicators)
    return out[0, 0]
